# baseline (speedup 1.0000x reference)
import jax
import jax.numpy as jnp
from jax.experimental import pallas as pl


def _copy_body(inputs_ref, buffer_ref, out_ref):
    i = pl.program_id(0)

    @pl.when(i == 0)
    def _():
        out_ref[...] = inputs_ref[...]

    @pl.when(i > 0)
    def _():
        out_ref[...] = buffer_ref[...]


def kernel(inputs, buffer):
    b, d = inputs.shape
    n_steps = buffer.shape[0]
    n_blocks = n_steps // b

    out_flat = pl.pallas_call(
        _copy_body,
        grid=(n_blocks,),
        in_specs=[
            pl.BlockSpec((b, d), lambda i: (0, 0)),
            pl.BlockSpec((b, d), lambda i: (jnp.maximum(i - 1, 0), 0)),
        ],
        out_specs=pl.BlockSpec((b, d), lambda i: (i, 0)),
        out_shape=jax.ShapeDtypeStruct((n_steps, d), inputs.dtype),
    )(inputs, buffer)
    return out_flat.reshape((b, n_steps // b, d))
